# baseline (speedup 1.0000x reference)
import functools

import jax
import jax.numpy as jnp
from jax import lax
from jax.experimental import pallas as pl
from jax.experimental.pallas import tpu as pltpu
from jax.experimental.pallas import tpu_sc as plsc

N = 10000
E = 160000
H = 128
L = 4
OE = 256
OL = 3
CUTOFF = 5.0

NP = 10240
NC = 2
NS = 16
CH = 128
EP = 163840
NSEG = 4
EH = EP // NSEG
NCHUNK = EH // CH
CPW = NCHUNK // (NC * NS)
CPS = NCHUNK // NS
BE_T = 1024
BN = 256
ROWS_SUB = NP // NS


def _swish(v):
    return v * jax.nn.sigmoid(v)


def _nodeprep_body(z_ref, emb_ref, ws_ref, wd_ref, h0_ref, hs_ref, hd_ref):
    z = z_ref[...]
    ids = lax.broadcasted_iota(jnp.int32, (BN, 128), 1)
    oh = (ids == z).astype(jnp.float32)
    h0 = jnp.dot(oh, emb_ref[...], preferred_element_type=jnp.float32)
    h0_ref[...] = h0
    hs_ref[...] = jnp.dot(h0, ws_ref[...], preferred_element_type=jnp.float32)
    hd_ref[...] = jnp.dot(h0, wd_ref[...], preferred_element_type=jnp.float32)


def _node_prep(zp, embp, Ws, Wd):
    f32 = jnp.float32
    return pl.pallas_call(
        _nodeprep_body,
        grid=(NP // BN,),
        in_specs=[
            pl.BlockSpec((BN, 1), lambda i: (i, 0)),
            pl.BlockSpec((128, H), lambda i: (0, 0)),
            pl.BlockSpec((H, H), lambda i: (0, 0)),
            pl.BlockSpec((H, H), lambda i: (0, 0)),
        ],
        out_specs=[
            pl.BlockSpec((BN, H), lambda i: (i, 0)),
            pl.BlockSpec((BN, H), lambda i: (i, 0)),
            pl.BlockSpec((BN, H), lambda i: (i, 0)),
        ],
        out_shape=[jax.ShapeDtypeStruct((NP, H), f32)] * 3,
    )(zp, embp, Ws, Wd)


def _gather_body(hs_hbm, hd_hbm, posflat_hbm, src2_hbm, dst2_hbm,
                 ges_hbm, d2_hbm,
                 pos_v, sidx_v, didx_v, bufa_v, bufb_v, d2a_v,
                 sga0, sga1, sgb0, sgb1, soa0, soa1):
    c = lax.axis_index("c")
    s = lax.axis_index("s")
    wid = s * NC + c
    lo = wid * CPW
    sga = (sga0, sga1)
    sgb = (sgb0, sgb1)
    soa = (soa0, soa1)
    pltpu.sync_copy(posflat_hbm, pos_v)
    pltpu.sync_copy(src2_hbm.at[wid], sidx_v)
    pltpu.sync_copy(dst2_hbm.at[wid], didx_v)

    ga = {}
    gb = {}
    oa = {}
    ga[0] = pltpu.async_copy(hs_hbm.at[sidx_v.at[0]], bufa_v.at[0], sga[0])
    gb[0] = pltpu.async_copy(hd_hbm.at[didx_v.at[0]], bufb_v.at[0], sgb[0])
    for j in range(CPW):
        p = j % 2
        if j + 1 < CPW:
            if j - 1 >= 0:
                oa[j - 1].wait()
            ga[j + 1] = pltpu.async_copy(
                hs_hbm.at[sidx_v.at[j + 1]], bufa_v.at[1 - p], sga[1 - p])
            gb[j + 1] = pltpu.async_copy(
                hd_hbm.at[didx_v.at[j + 1]], bufb_v.at[1 - p], sgb[1 - p])
        for k in range(CH // 16):
            si = sidx_v[j, pl.ds(k * 16, 16)] * 4
            di = didx_v[j, pl.ds(k * 16, 16)] * 4
            acc = jnp.zeros((16,), jnp.float32)
            for coord in range(3):
                a = plsc.load_gather(pos_v, [si + coord])
                bb = plsc.load_gather(pos_v, [di + coord])
                t = a - bb
                acc = acc + t * t
            d2a_v[pl.ds(j * CH + k * 16, 16)] = acc
        base = (lo + j) * CH
        ga[j].wait()
        gb[j].wait()

        def _sum_row(r, carry):
            for k in range(H // 16):
                bufa_v[p, r, pl.ds(k * 16, 16)] = (
                    bufa_v[p, r, pl.ds(k * 16, 16)]
                    + bufb_v[p, r, pl.ds(k * 16, 16)])
            return carry

        lax.fori_loop(0, CH, _sum_row, 0)
        oa[j] = pltpu.async_copy(bufa_v.at[p], ges_hbm.at[pl.ds(base, CH)],
                                 soa[p])
    oa[CPW - 1].wait()
    oa[CPW - 2].wait()
    pltpu.sync_copy(d2a_v, d2_hbm.at[pl.ds(lo * CH, CPW * CH)])


def _edge_gather(hs, hd, posflat, src2, dst2):
    f32 = jnp.float32
    i32 = jnp.int32
    mesh = plsc.VectorSubcoreMesh(core_axis_name="c", subcore_axis_name="s")
    k = pl.kernel(
        _gather_body,
        mesh=mesh,
        out_type=[
            jax.ShapeDtypeStruct((EH, H), f32),
            jax.ShapeDtypeStruct((EH,), f32),
        ],
        scratch_types=[
            pltpu.VMEM((NP * 4,), f32),
            pltpu.VMEM((CPW, CH), i32),
            pltpu.VMEM((CPW, CH), i32),
            pltpu.VMEM((2, CH, H), f32),
            pltpu.VMEM((2, CH, H), f32),
            pltpu.VMEM((CPW * CH,), f32),
        ] + [pltpu.SemaphoreType.DMA] * 6,
        compiler_params=pltpu.CompilerParams(needs_layout_passes=False),
    )
    return k(hs, hd, posflat, src2, dst2)


def _edge_chain_body(ges_ref, d2_ref, wr3_ref, b_ref,
                     wrl_ref, wmsg_ref, m_ref):
    d2 = d2_ref[...]
    x = jnp.sqrt(d2 + 1e-12) * (1.0 / CUTOFF)
    env = 1.0 / x - 28.0 * x**5 + 48.0 * x**6 - 21.0 * x**7
    env = jnp.where(x < 1.0, env, 0.0)
    n = lax.broadcasted_iota(jnp.int32, (BE_T, 8), 1).astype(jnp.float32) + 1.0
    rbf = jnp.where(n <= 6.0, env * jnp.sin(n * jnp.pi * x), 0.0)
    m = (ges_ref[...]
         + jnp.dot(rbf, wr3_ref[...], preferred_element_type=jnp.float32)
         + b_ref[...])
    m = _swish(m)
    for l in range(L):
        rh = jnp.dot(rbf, wrl_ref[l], preferred_element_type=jnp.float32)
        t = jnp.dot(m.astype(jnp.bfloat16), wmsg_ref[l],
                    preferred_element_type=jnp.float32)
        m = _swish(t) * rh + m
        m_ref[l] = m


def _edge_chain(ges, d2, Wr3, b, Wrlp, Wmsg):
    f32 = jnp.float32
    return pl.pallas_call(
        _edge_chain_body,
        grid=(EH // BE_T,),
        in_specs=[
            pl.BlockSpec((BE_T, H), lambda i: (i, 0)),
            pl.BlockSpec((BE_T, 1), lambda i: (i, 0)),
            pl.BlockSpec((8, H), lambda i: (0, 0)),
            pl.BlockSpec((1, H), lambda i: (0, 0)),
            pl.BlockSpec((L, 8, H), lambda i: (0, 0, 0)),
            pl.BlockSpec((L, H, H), lambda i: (0, 0, 0)),
        ],
        out_specs=pl.BlockSpec((L, BE_T, H), lambda i: (0, i, 0)),
        out_shape=jax.ShapeDtypeStruct((L, EH, H), f32),
    )(ges, d2, Wr3, b, Wrlp, Wmsg)


def _scatter_body(m_hbm, dst2_hbm, zeros_hbm, agg_hbm,
                  idx_v, rm_v, sl0, sl1, ss0, ss1, acc_shared):
    c = lax.axis_index("c")
    s = lax.axis_index("s")
    lo = s * CPS
    sl = (sl0, sl1)
    ss = (ss0, ss1)
    pltpu.sync_copy(dst2_hbm.at[s], idx_v)
    for li in range(L // NC):
        l = c * (L // NC) + li
        pltpu.sync_copy(zeros_hbm.at[l, pl.ds(s * ROWS_SUB, ROWS_SUB)],
                        acc_shared.at[pl.ds(s * ROWS_SUB, ROWS_SUB)])
        plsc.subcore_barrier()
        ld = {}
        sc = {}
        ld[0] = pltpu.async_copy(m_hbm.at[l, pl.ds(lo * CH, CH)],
                                 rm_v.at[0], sl[0])
        for j in range(CPS):
            p = j % 2
            ld[j].wait()
            if j + 1 < CPS:
                if j - 1 >= 0:
                    sc[j - 1].wait()
                ld[j + 1] = pltpu.async_copy(
                    m_hbm.at[l, pl.ds((lo + j + 1) * CH, CH)],
                    rm_v.at[1 - p], sl[1 - p])
            sc[j] = pltpu.async_copy(rm_v.at[p], acc_shared.at[idx_v.at[j]],
                                     ss[p], add=True)
        sc[CPS - 1].wait()
        sc[CPS - 2].wait()
        plsc.subcore_barrier()
        pltpu.sync_copy(acc_shared.at[pl.ds(s * ROWS_SUB, ROWS_SUB)],
                        agg_hbm.at[l, pl.ds(s * ROWS_SUB, ROWS_SUB)])
        plsc.subcore_barrier()


def _edge_scatter(m_all, dst2, zeros_np):
    f32 = jnp.float32
    mesh = plsc.VectorSubcoreMesh(core_axis_name="c", subcore_axis_name="s")
    k = pl.kernel(
        _scatter_body,
        mesh=mesh,
        out_type=[jax.ShapeDtypeStruct((L, NP, H), f32)],
        scratch_types=[
            pltpu.VMEM((CPS, CH), jnp.int32),
            pltpu.VMEM((2, CH, H), f32),
            pltpu.SemaphoreType.DMA,
            pltpu.SemaphoreType.DMA,
            pltpu.SemaphoreType.DMA,
            pltpu.SemaphoreType.DMA,
            pltpu.VMEM_SHARED((NP, H), f32),
        ],
    )
    return k(m_all, dst2, zeros_np)[0]


def _finish_body(h0_ref, agg_ref, wupd_ref, wo1_ref, wmid_ref,
                 wof_ref, out_ref):
    i = pl.program_id(0)
    h = h0_ref[...]
    for l in range(L):
        t = jnp.dot(agg_ref[l], wupd_ref[l],
                    preferred_element_type=jnp.float32)
        h = h + _swish(t)
    o = _swish(jnp.dot(h, wo1_ref[...], preferred_element_type=jnp.float32))
    for l in range(OL):
        o = _swish(jnp.dot(o, wmid_ref[l], preferred_element_type=jnp.float32))
    e = jnp.dot(o, wof_ref[...], preferred_element_type=jnp.float32)
    rid = lax.broadcasted_iota(jnp.int32, (BN, 1), 0) + i * BN
    part = jnp.sum(jnp.where(rid < N, e, 0.0)).reshape(1, 1)
    prev = jnp.where(i == 0, jnp.zeros((1, 1), jnp.float32), out_ref[...])
    out_ref[...] = prev + part


def _finish(h0, agg, Wupd, Wo1, Wmid, Wof):
    f32 = jnp.float32
    return pl.pallas_call(
        _finish_body,
        grid=(NP // BN,),
        in_specs=[
            pl.BlockSpec((BN, H), lambda i: (i, 0)),
            pl.BlockSpec((L, BN, H), lambda i: (0, i, 0)),
            pl.BlockSpec((L, H, H), lambda i: (0, 0, 0)),
            pl.BlockSpec((H, OE), lambda i: (0, 0)),
            pl.BlockSpec((OL, OE, OE), lambda i: (0, 0, 0)),
            pl.BlockSpec((OE, 1), lambda i: (0, 0)),
        ],
        out_specs=pl.BlockSpec((1, 1), lambda i: (0, 0)),
        out_shape=jax.ShapeDtypeStruct((1, 1), f32),
    )(h0, agg, Wupd, Wo1, Wmid, Wof)


def kernel(z, pos, edge_index, node_emb, W_init, b_init, W_rbf1, W_rbf2,
           W_msg, W_upd, W_out1, W_out_mid, W_out_final):
    f32 = jnp.float32
    src3 = jnp.pad(edge_index[0].astype(jnp.int32), (0, EP - E)
                   ).reshape(NSEG, NC * NS, CPW, CH)
    dst3 = jnp.pad(edge_index[1].astype(jnp.int32), (0, EP - E),
                   constant_values=N).reshape(NSEG, NC * NS, CPW, CH)
    zp = jnp.pad(z.astype(jnp.int32), (0, NP - N)).reshape(NP, 1)
    posflat = jnp.zeros((NP, 4), f32).at[:N, :3].set(pos).reshape(NP * 4)
    embp = jnp.zeros((128, H), f32).at[:node_emb.shape[0]].set(node_emb)
    Ws = W_init[:H]
    Wd = W_init[H:2 * H]
    Wr3 = jnp.zeros((8, H), f32).at[:6].set(W_init[2 * H:])
    Wrlp = jnp.zeros((L, 8, H), f32).at[:, :6].set(
        jnp.einsum('lrb,lbh->lrh', W_rbf1, W_rbf2))
    b = b_init.reshape(1, H)

    h0, hs, hd = _node_prep(zp, embp, Ws, Wd)
    Wmsg16 = W_msg.astype(jnp.bfloat16)
    Wupd16 = W_upd.astype(jnp.bfloat16)
    m_halves = []
    for hf in range(NSEG):
        ges, d2 = _edge_gather(hs, hd, posflat, src3[hf], dst3[hf])
        m_all = _edge_chain(ges, d2.reshape(EH, 1), Wr3, b, Wrlp, Wmsg16)
        m_halves.append((m_all, dst3[hf].reshape(NS, CPS, CH)))
    acc = jnp.zeros((L, NP, H), f32)
    for m_all, dsc in m_halves:
        acc = _edge_scatter(m_all, dsc, acc)
    out = _finish(h0, acc, Wupd16, W_out1, W_out_mid, W_out_final)
    return out[0, 0]


# speedup vs baseline: 1.0929x; 1.0929x over previous
import functools

import jax
import jax.numpy as jnp
from jax import lax
from jax.experimental import pallas as pl
from jax.experimental.pallas import tpu as pltpu
from jax.experimental.pallas import tpu_sc as plsc

N = 10000
E = 160000
H = 128
L = 4
OE = 256
OL = 3
CUTOFF = 5.0

NP = 10240
NC = 2
NS = 16
CH = 128
EP = 163840
NSEG = 2
EH = EP // NSEG
NCHUNK = EH // CH
CPW = NCHUNK // (NC * NS)
CPS = NCHUNK // NS
BE_T = 1024
BN = 256
ROWS_SUB = NP // NS


def _swish(v):
    return v * jax.nn.sigmoid(v)


def _nodeprep_body(z_ref, emb_ref, ws_ref, wd_ref, h0_ref, hs_ref, hd_ref):
    z = z_ref[...]
    ids = lax.broadcasted_iota(jnp.int32, (BN, 128), 1)
    oh = (ids == z).astype(jnp.float32)
    h0 = jnp.dot(oh, emb_ref[...], preferred_element_type=jnp.float32)
    h0_ref[...] = h0
    hs_ref[...] = jnp.dot(h0, ws_ref[...], preferred_element_type=jnp.float32)
    hd_ref[...] = jnp.dot(h0, wd_ref[...], preferred_element_type=jnp.float32)


def _node_prep(zp, embp, Ws, Wd):
    f32 = jnp.float32
    return pl.pallas_call(
        _nodeprep_body,
        grid=(NP // BN,),
        in_specs=[
            pl.BlockSpec((BN, 1), lambda i: (i, 0)),
            pl.BlockSpec((128, H), lambda i: (0, 0)),
            pl.BlockSpec((H, H), lambda i: (0, 0)),
            pl.BlockSpec((H, H), lambda i: (0, 0)),
        ],
        out_specs=[
            pl.BlockSpec((BN, H), lambda i: (i, 0)),
            pl.BlockSpec((BN, H), lambda i: (i, 0)),
            pl.BlockSpec((BN, H), lambda i: (i, 0)),
        ],
        out_shape=[jax.ShapeDtypeStruct((NP, H), f32)] * 3,
    )(zp, embp, Ws, Wd)


def _gather_body(hs_hbm, hd_hbm, posflat_hbm, src2_hbm, dst2_hbm,
                 ges_hbm, d2_hbm,
                 pos_v, sidx_v, didx_v, bufa_v, bufb_v, d2a_v,
                 sga0, sga1, sgb0, sgb1, soa0, soa1):
    c = lax.axis_index("c")
    s = lax.axis_index("s")
    wid = s * NC + c
    lo = wid * CPW
    sga = (sga0, sga1)
    sgb = (sgb0, sgb1)
    soa = (soa0, soa1)
    pltpu.sync_copy(posflat_hbm, pos_v)
    pltpu.sync_copy(src2_hbm.at[wid], sidx_v)
    pltpu.sync_copy(dst2_hbm.at[wid], didx_v)

    ga = {}
    gb = {}
    oa = {}
    ga[0] = pltpu.async_copy(hs_hbm.at[sidx_v.at[0]], bufa_v.at[0], sga[0])
    gb[0] = pltpu.async_copy(hd_hbm.at[didx_v.at[0]], bufb_v.at[0], sgb[0])
    for j in range(CPW):
        p = j % 2
        if j + 1 < CPW:
            if j - 1 >= 0:
                oa[j - 1].wait()
            ga[j + 1] = pltpu.async_copy(
                hs_hbm.at[sidx_v.at[j + 1]], bufa_v.at[1 - p], sga[1 - p])
            gb[j + 1] = pltpu.async_copy(
                hd_hbm.at[didx_v.at[j + 1]], bufb_v.at[1 - p], sgb[1 - p])
        for k in range(CH // 16):
            si = sidx_v[j, pl.ds(k * 16, 16)] * 4
            di = didx_v[j, pl.ds(k * 16, 16)] * 4
            acc = jnp.zeros((16,), jnp.float32)
            for coord in range(3):
                a = plsc.load_gather(pos_v, [si + coord])
                bb = plsc.load_gather(pos_v, [di + coord])
                t = a - bb
                acc = acc + t * t
            d2a_v[pl.ds(j * CH + k * 16, 16)] = acc
        base = (lo + j) * CH
        ga[j].wait()
        gb[j].wait()

        def _sum_row(r, carry):
            for k in range(H // 16):
                bufa_v[p, r, pl.ds(k * 16, 16)] = (
                    bufa_v[p, r, pl.ds(k * 16, 16)]
                    + bufb_v[p, r, pl.ds(k * 16, 16)])
            return carry

        lax.fori_loop(0, CH, _sum_row, 0)
        oa[j] = pltpu.async_copy(bufa_v.at[p], ges_hbm.at[pl.ds(base, CH)],
                                 soa[p])
    oa[CPW - 1].wait()
    oa[CPW - 2].wait()
    pltpu.sync_copy(d2a_v, d2_hbm.at[pl.ds(lo * CH, CPW * CH)])


def _edge_gather(hs, hd, posflat, src2, dst2):
    f32 = jnp.float32
    i32 = jnp.int32
    mesh = plsc.VectorSubcoreMesh(core_axis_name="c", subcore_axis_name="s")
    k = pl.kernel(
        _gather_body,
        mesh=mesh,
        out_type=[
            jax.ShapeDtypeStruct((EH, H), f32),
            jax.ShapeDtypeStruct((EH,), f32),
        ],
        scratch_types=[
            pltpu.VMEM((NP * 4,), f32),
            pltpu.VMEM((CPW, CH), i32),
            pltpu.VMEM((CPW, CH), i32),
            pltpu.VMEM((2, CH, H), f32),
            pltpu.VMEM((2, CH, H), f32),
            pltpu.VMEM((CPW * CH,), f32),
        ] + [pltpu.SemaphoreType.DMA] * 6,
        compiler_params=pltpu.CompilerParams(needs_layout_passes=False),
    )
    return k(hs, hd, posflat, src2, dst2)


def _edge_chain_body(ges_ref, d2_ref, wr3_ref, b_ref,
                     wrl_ref, wmsg_ref, m_ref):
    d2 = d2_ref[...]
    x = jnp.sqrt(d2 + 1e-12) * (1.0 / CUTOFF)
    env = 1.0 / x - 28.0 * x**5 + 48.0 * x**6 - 21.0 * x**7
    env = jnp.where(x < 1.0, env, 0.0)
    n = lax.broadcasted_iota(jnp.int32, (BE_T, 8), 1).astype(jnp.float32) + 1.0
    rbf = jnp.where(n <= 6.0, env * jnp.sin(n * jnp.pi * x), 0.0)
    m = (ges_ref[...]
         + jnp.dot(rbf, wr3_ref[...], preferred_element_type=jnp.float32)
         + b_ref[...])
    m = _swish(m)
    for l in range(L):
        rh = jnp.dot(rbf, wrl_ref[l], preferred_element_type=jnp.float32)
        t = jnp.dot(m.astype(jnp.bfloat16), wmsg_ref[l],
                    preferred_element_type=jnp.float32)
        m = _swish(t) * rh + m
        m_ref[l] = m


def _edge_chain(ges, d2, Wr3, b, Wrlp, Wmsg):
    f32 = jnp.float32
    return pl.pallas_call(
        _edge_chain_body,
        grid=(EH // BE_T,),
        in_specs=[
            pl.BlockSpec((BE_T, H), lambda i: (i, 0)),
            pl.BlockSpec((BE_T, 1), lambda i: (i, 0)),
            pl.BlockSpec((8, H), lambda i: (0, 0)),
            pl.BlockSpec((1, H), lambda i: (0, 0)),
            pl.BlockSpec((L, 8, H), lambda i: (0, 0, 0)),
            pl.BlockSpec((L, H, H), lambda i: (0, 0, 0)),
        ],
        out_specs=pl.BlockSpec((L, BE_T, H), lambda i: (0, i, 0)),
        out_shape=jax.ShapeDtypeStruct((L, EH, H), f32),
    )(ges, d2, Wr3, b, Wrlp, Wmsg)


def _scatter_body(m_hbm, dst2_hbm, zeros_hbm, agg_hbm,
                  idx_v, rm_v, sl0, sl1, ss0, ss1, acc_shared):
    c = lax.axis_index("c")
    s = lax.axis_index("s")
    lo = s * CPS
    sl = (sl0, sl1)
    ss = (ss0, ss1)
    pltpu.sync_copy(dst2_hbm.at[s], idx_v)
    for li in range(L // NC):
        l = c * (L // NC) + li
        pltpu.sync_copy(zeros_hbm.at[l, pl.ds(s * ROWS_SUB, ROWS_SUB)],
                        acc_shared.at[pl.ds(s * ROWS_SUB, ROWS_SUB)])
        plsc.subcore_barrier()
        ld = {}
        sc = {}
        ld[0] = pltpu.async_copy(m_hbm.at[l, pl.ds(lo * CH, CH)],
                                 rm_v.at[0], sl[0])
        for j in range(CPS):
            p = j % 2
            ld[j].wait()
            if j + 1 < CPS:
                if j - 1 >= 0:
                    sc[j - 1].wait()
                ld[j + 1] = pltpu.async_copy(
                    m_hbm.at[l, pl.ds((lo + j + 1) * CH, CH)],
                    rm_v.at[1 - p], sl[1 - p])
            sc[j] = pltpu.async_copy(rm_v.at[p], acc_shared.at[idx_v.at[j]],
                                     ss[p], add=True)
        sc[CPS - 1].wait()
        sc[CPS - 2].wait()
        plsc.subcore_barrier()
        pltpu.sync_copy(acc_shared.at[pl.ds(s * ROWS_SUB, ROWS_SUB)],
                        agg_hbm.at[l, pl.ds(s * ROWS_SUB, ROWS_SUB)])
        plsc.subcore_barrier()


def _edge_scatter(m_all, dst2, zeros_np):
    f32 = jnp.float32
    mesh = plsc.VectorSubcoreMesh(core_axis_name="c", subcore_axis_name="s")
    k = pl.kernel(
        _scatter_body,
        mesh=mesh,
        out_type=[jax.ShapeDtypeStruct((L, NP, H), f32)],
        scratch_types=[
            pltpu.VMEM((CPS, CH), jnp.int32),
            pltpu.VMEM((2, CH, H), f32),
            pltpu.SemaphoreType.DMA,
            pltpu.SemaphoreType.DMA,
            pltpu.SemaphoreType.DMA,
            pltpu.SemaphoreType.DMA,
            pltpu.VMEM_SHARED((NP, H), f32),
        ],
    )
    return k(m_all, dst2, zeros_np)[0]


def _finish_body(h0_ref, agg_ref, wupd_ref, wo1_ref, wmid_ref,
                 wof_ref, out_ref):
    i = pl.program_id(0)
    h = h0_ref[...]
    for l in range(L):
        t = jnp.dot(agg_ref[l], wupd_ref[l],
                    preferred_element_type=jnp.float32)
        h = h + _swish(t)
    o = _swish(jnp.dot(h, wo1_ref[...], preferred_element_type=jnp.float32))
    for l in range(OL):
        o = _swish(jnp.dot(o, wmid_ref[l], preferred_element_type=jnp.float32))
    e = jnp.dot(o, wof_ref[...], preferred_element_type=jnp.float32)
    rid = lax.broadcasted_iota(jnp.int32, (BN, 1), 0) + i * BN
    part = jnp.sum(jnp.where(rid < N, e, 0.0)).reshape(1, 1)
    prev = jnp.where(i == 0, jnp.zeros((1, 1), jnp.float32), out_ref[...])
    out_ref[...] = prev + part


def _finish(h0, agg, Wupd, Wo1, Wmid, Wof):
    f32 = jnp.float32
    return pl.pallas_call(
        _finish_body,
        grid=(NP // BN,),
        in_specs=[
            pl.BlockSpec((BN, H), lambda i: (i, 0)),
            pl.BlockSpec((L, BN, H), lambda i: (0, i, 0)),
            pl.BlockSpec((L, H, H), lambda i: (0, 0, 0)),
            pl.BlockSpec((H, OE), lambda i: (0, 0)),
            pl.BlockSpec((OL, OE, OE), lambda i: (0, 0, 0)),
            pl.BlockSpec((OE, 1), lambda i: (0, 0)),
        ],
        out_specs=pl.BlockSpec((1, 1), lambda i: (0, 0)),
        out_shape=jax.ShapeDtypeStruct((1, 1), f32),
    )(h0, agg, Wupd, Wo1, Wmid, Wof)


def kernel(z, pos, edge_index, node_emb, W_init, b_init, W_rbf1, W_rbf2,
           W_msg, W_upd, W_out1, W_out_mid, W_out_final):
    f32 = jnp.float32
    src3 = jnp.pad(edge_index[0].astype(jnp.int32), (0, EP - E)
                   ).reshape(NSEG, NC * NS, CPW, CH)
    dst3 = jnp.pad(edge_index[1].astype(jnp.int32), (0, EP - E),
                   constant_values=N).reshape(NSEG, NC * NS, CPW, CH)
    zp = jnp.pad(z.astype(jnp.int32), (0, NP - N)).reshape(NP, 1)
    posflat = jnp.zeros((NP, 4), f32).at[:N, :3].set(pos).reshape(NP * 4)
    embp = jnp.zeros((128, H), f32).at[:node_emb.shape[0]].set(node_emb)
    Ws = W_init[:H]
    Wd = W_init[H:2 * H]
    Wr3 = jnp.zeros((8, H), f32).at[:6].set(W_init[2 * H:])
    Wrlp = jnp.zeros((L, 8, H), f32).at[:, :6].set(
        jnp.einsum('lrb,lbh->lrh', W_rbf1, W_rbf2))
    b = b_init.reshape(1, H)

    h0, hs, hd = _node_prep(zp, embp, Ws, Wd)
    Wmsg16 = W_msg.astype(jnp.bfloat16)
    Wupd16 = W_upd.astype(jnp.bfloat16)
    m_halves = []
    for hf in range(NSEG):
        ges, d2 = _edge_gather(hs, hd, posflat, src3[hf], dst3[hf])
        m_all = _edge_chain(ges, d2.reshape(EH, 1), Wr3, b, Wrlp, Wmsg16)
        m_halves.append((m_all, dst3[hf].reshape(NS, CPS, CH)))
    acc = jnp.zeros((L, NP, H), f32)
    for m_all, dsc in m_halves:
        acc = _edge_scatter(m_all, dsc, acc)
    out = _finish(h0, acc, Wupd16, W_out1, W_out_mid, W_out_final)
    return out[0, 0]
